# baseline (speedup 1.0000x reference)
import jax
import jax.numpy as jnp
from jax.experimental import pallas as pl
from jax.experimental.pallas import tpu as pltpu


def _semantic_combine_kernel(h_ap_ref, h_sp_ref, fc_w_ref, fc_b_ref, attn_sem_ref, out_ref):
    h_ap = h_ap_ref[...]
    h_sp = h_sp_ref[...]
    fc_w = fc_w_ref[...]
    fc_b = fc_b_ref[...]
    attn_sem = attn_sem_ref[...]
    t_ap = jnp.tanh(jax.lax.dot_general(h_ap, fc_w, (((1,), (1,)), ((), ())),
                                        preferred_element_type=jnp.float32) + fc_b[None, :])
    t_sp = jnp.tanh(jax.lax.dot_general(h_sp, fc_w, (((1,), (1,)), ((), ())),
                                        preferred_element_type=jnp.float32) + fc_b[None, :])
    w_ap = jnp.sum(jnp.mean(t_ap, axis=0) * attn_sem[0])
    w_sp = jnp.sum(jnp.mean(t_sp, axis=0) * attn_sem[0])
    m = jnp.maximum(w_ap, w_sp)
    e_ap = jnp.exp(w_ap - m)
    e_sp = jnp.exp(w_sp - m)
    denom = e_ap + e_sp
    b_ap = e_ap / denom
    b_sp = e_sp / denom
    out_ref[...] = b_ap * h_ap + b_sp * h_sp


def _semantic_combine(h_ap, h_sp, fc_w, fc_b, attn_sem):
    n, d = h_ap.shape
    return pl.pallas_call(
        _semantic_combine_kernel,
        out_shape=jax.ShapeDtypeStruct((n, d), jnp.float32),
    )(h_ap, h_sp, fc_w, fc_b, attn_sem)


def _gat_conv(feat_src, feat_dst, edge_index, attn_l, attn_r, num_dst):
    src = edge_index[0]
    dst = edge_index[1]
    el = (feat_src * attn_l).sum(axis=-1)
    er = (feat_dst * attn_r).sum(axis=-1)
    e = el[src] + er[dst]
    e = jnp.where(e >= 0, e, 0.01 * e)
    emax = jax.ops.segment_max(e, dst, num_segments=num_dst)
    emax = jnp.where(jnp.isfinite(emax), emax, 0.0)
    ee = jnp.exp(e - emax[dst])
    denom = jax.ops.segment_sum(ee, dst, num_segments=num_dst)
    a = ee / denom[dst]
    m = feat_src[src] * a[:, None]
    out = jax.ops.segment_sum(m, dst, num_segments=num_dst)
    return jax.nn.elu(out)


def kernel(feat_author, feat_subject, feat_paper, edge_index_ap, edge_index_sp,
           attn_l_ap, attn_r_ap, attn_l_sp, attn_r_sp, fc_w, fc_b, attn_sem):
    num_dst = feat_paper.shape[0]
    h_ap = _gat_conv(feat_author, feat_paper, edge_index_ap, attn_l_ap, attn_r_ap, num_dst)
    h_sp = _gat_conv(feat_subject, feat_paper, edge_index_sp, attn_l_sp, attn_r_sp, num_dst)
    return _semantic_combine(h_ap, h_sp, fc_w, fc_b, attn_sem)


# speedup vs baseline: 25.0294x; 25.0294x over previous
import dataclasses
import functools

import jax
import jax.numpy as jnp
from jax import lax
from jax.experimental import pallas as pl
from jax.experimental.pallas import tpu as pltpu
from jax.experimental.pallas import tpu_sc as plsc

N = 10000
E = 320000
D = 128
DW = 16
C = 128
NCHUNK = E // C
NWORK = 32
BASE_CHUNKS = NCHUNK // NWORK
EXTRA = NCHUNK - BASE_CHUNKS * NWORK
B = 4
NBATCH = (BASE_CHUNKS + 1 + B - 1) // B
PAD_CHUNKS = NCHUNK + B

_SC_COMPILER_PARAMS = pltpu.CompilerParams(use_tc_tiling_on_sc=False)
if "needs_layout_passes" in pltpu.CompilerParams.__dataclass_fields__:
    _SC_COMPILER_PARAMS = dataclasses.replace(_SC_COMPILER_PARAMS, needs_layout_passes=False)

ROWS_PER_TILE = 624
SLAB = 104
TAIL_ROWS = N - 16 * ROWS_PER_TILE


def _scalar_table_body(fa_ref, fs_ref, fp_ref, lap_ref, rap_ref, lsp_ref, rsp_ref, out_ref):
    fa = fa_ref[...]
    fs = fs_ref[...]
    fp = fp_ref[...]
    el_ap = jnp.sum(fa * lap_ref[...][0][None, :], axis=1)
    er_ap = jnp.sum(fp * rap_ref[...][0][None, :], axis=1)
    el_sp = jnp.sum(fs * lsp_ref[...][0][None, :], axis=1)
    er_sp = jnp.sum(fp * rsp_ref[...][0][None, :], axis=1)
    out_ref[...] = jnp.stack([el_ap, er_ap, el_sp, er_sp], axis=0)


def _scalar_table(feat_author, feat_subject, feat_paper, attn_l_ap, attn_r_ap, attn_l_sp, attn_r_sp):
    return pl.pallas_call(
        _scalar_table_body,
        out_shape=jax.ShapeDtypeStruct((4, N), jnp.float32),
    )(feat_author, feat_subject, feat_paper, attn_l_ap, attn_r_ap, attn_l_sp, attn_r_sp)


def _sc_gat(scal, src_ap, dst_ap, src_sp, dst_sp, feat_a, feat_s):
    mesh = plsc.VectorSubcoreMesh(core_axis_name="c", subcore_axis_name="s")

    @functools.partial(
        pl.kernel,
        out_type=[
            jax.ShapeDtypeStruct((2, N, D), jnp.float32),
            jax.ShapeDtypeStruct((2, N, DW), jnp.float32),
            jax.ShapeDtypeStruct((2, N, D), jnp.float32),
            jax.ShapeDtypeStruct((2, N, DW), jnp.float32),
        ],
        mesh=mesh,
        scratch_types=[
            pltpu.VMEM((N,), jnp.float32),
            pltpu.VMEM((N,), jnp.float32),
            pltpu.VMEM((B, C), jnp.int32),
            pltpu.VMEM((B, C), jnp.int32),
            pltpu.VMEM((C, D), jnp.float32),
            pltpu.VMEM((C, DW), jnp.float32),
            pltpu.VMEM((C,), jnp.float32),
            pltpu.VMEM_SHARED((N, D), jnp.float32),
            pltpu.VMEM_SHARED((N, DW), jnp.float32),
            pltpu.SemaphoreType.DMA,
        ],
        compiler_params=_SC_COMPILER_PARAMS,
    )
    def kern(scal_hbm, src_ap_hbm, dst_ap_hbm, src_sp_hbm, dst_sp_hbm,
             feat_a_hbm, feat_s_hbm, out_ap_hbm, den_ap_hbm, out_sp_hbm, den_sp_hbm,
             el_t, er_t, src_t, dst_t, rows_g, p_rows, p_col, acc, accd, sem):
        cid = lax.axis_index("c")
        sid = lax.axis_index("s")
        wid = sid * 2 + cid
        n_my = BASE_CHUNKS + jnp.where(wid < EXTRA, 1, 0)
        cstart = wid * BASE_CHUNKS + jnp.minimum(wid, EXTRA)

        col_iota = lax.iota(jnp.int32, 16)
        denom_mask = jnp.where(col_iota == 0, 1.0, 0.0).astype(jnp.float32)
        zz = jnp.zeros((16,), jnp.float32)

        def zero_local():
            @pl.loop(0, C)
            def _(r):
                for g in range(D // 16):
                    rows_g[r, pl.ds(16 * g, 16)] = zz
                p_rows[r, :] = zz

        def zero_acc():
            for k in range(ROWS_PER_TILE // SLAB):
                r0 = sid * ROWS_PER_TILE + k * SLAB
                pltpu.sync_copy(rows_g.at[pl.ds(0, SLAB), :], acc.at[pl.ds(r0, SLAB), :])
                pltpu.sync_copy(p_rows.at[pl.ds(0, SLAB), :], accd.at[pl.ds(r0, SLAB), :])

            @pl.when(sid == 15)
            def _():
                r0 = 16 * ROWS_PER_TILE
                pltpu.sync_copy(rows_g.at[pl.ds(0, TAIL_ROWS), :], acc.at[pl.ds(r0, TAIL_ROWS), :])
                pltpu.sync_copy(p_rows.at[pl.ds(0, TAIL_ROWS), :], accd.at[pl.ds(r0, TAIL_ROWS), :])

        def drain(out_hbm, den_hbm):
            for k in range(ROWS_PER_TILE // SLAB):
                r0 = sid * ROWS_PER_TILE + k * SLAB
                pltpu.sync_copy(acc.at[pl.ds(r0, SLAB), :], out_hbm.at[cid].at[pl.ds(r0, SLAB), :])
                pltpu.sync_copy(accd.at[pl.ds(r0, SLAB), :], den_hbm.at[cid].at[pl.ds(r0, SLAB), :])

            @pl.when(sid == 15)
            def _():
                r0 = 16 * ROWS_PER_TILE
                pltpu.sync_copy(acc.at[pl.ds(r0, TAIL_ROWS), :], out_hbm.at[cid].at[pl.ds(r0, TAIL_ROWS), :])
                pltpu.sync_copy(accd.at[pl.ds(r0, TAIL_ROWS), :], den_hbm.at[cid].at[pl.ds(r0, TAIL_ROWS), :])

        def run_relation(src_hbm, dst_hbm, feat_hbm, el_row, er_row):
            pltpu.sync_copy(scal_hbm.at[el_row], el_t)
            pltpu.sync_copy(scal_hbm.at[er_row], er_t)

            @pl.loop(0, NBATCH)
            def _(b):
                b0 = cstart + b * B
                pltpu.sync_copy(src_hbm.at[pl.ds(b0, B), :], src_t)
                pltpu.sync_copy(dst_hbm.at[pl.ds(b0, B), :], dst_t)
                jcount = jnp.minimum(n_my - b * B, B)

                @pl.loop(0, jcount)
                def _(j):
                    pltpu.async_copy(feat_hbm.at[src_t.at[j]], rows_g, sem).wait()
                    for g in range(C // 16):
                        sv = src_t[j, pl.ds(16 * g, 16)]
                        dv = dst_t[j, pl.ds(16 * g, 16)]
                        e = plsc.load_gather(el_t, [sv]) + plsc.load_gather(er_t, [dv])
                        e = jnp.where(e >= 0.0, e, 0.01 * e)
                        p_col[pl.ds(16 * g, 16)] = jnp.exp(e)

                    @pl.loop(0, C)
                    def _(r):
                        psp = plsc.load_gather(p_col, [jnp.full((16,), r, jnp.int32)])
                        for g in range(D // 16):
                            rows_g[r, pl.ds(16 * g, 16)] = rows_g[r, pl.ds(16 * g, 16)] * psp
                        p_rows[r, :] = psp * denom_mask

                    pltpu.sync_copy(rows_g, acc.at[dst_t.at[j]], add=True)
                    pltpu.sync_copy(p_rows, accd.at[dst_t.at[j]], add=True)

        zero_local()
        zero_acc()
        plsc.subcore_barrier()
        run_relation(src_ap_hbm, dst_ap_hbm, feat_a_hbm, 0, 1)
        plsc.subcore_barrier()
        drain(out_ap_hbm, den_ap_hbm)
        zero_local()
        zero_acc()
        plsc.subcore_barrier()
        run_relation(src_sp_hbm, dst_sp_hbm, feat_s_hbm, 2, 3)
        plsc.subcore_barrier()
        drain(out_sp_hbm, den_sp_hbm)

    return kern(scal, src_ap, dst_ap, src_sp, dst_sp, feat_a, feat_s)


ROWS_TC = 1000
GRID_TC = N // ROWS_TC


def _post_body(ap_ref, dap_ref, sp_ref, dsp_ref, fcw_ref, fcb_ref,
               h_ap_ref, h_sp_ref, tsum_ref):
    step = pl.program_id(0)

    @pl.when(step == 0)
    def _():
        tsum_ref[...] = jnp.zeros_like(tsum_ref)

    fcw = fcw_ref[...]
    fcb = fcb_ref[...]
    for m, (part_ref, den_ref, h_ref) in enumerate(
            ((ap_ref, dap_ref, h_ap_ref), (sp_ref, dsp_ref, h_sp_ref))):
        num = part_ref[...][0] + part_ref[...][1]
        denf = den_ref[...][0] + den_ref[...][1]
        den = denf[:, 0:1]
        h = jnp.where(den > 0.0, num / jnp.where(den > 0.0, den, 1.0), 0.0)
        h = jnp.where(h > 0.0, h, jnp.exp(h) - 1.0)
        h_ref[...] = h
        t = jnp.tanh(
            jax.lax.dot_general(h, fcw, (((1,), (1,)), ((), ())),
                                preferred_element_type=jnp.float32) + fcb[None, :])
        tsum_ref[pl.ds(m, 1), :] += jnp.sum(t, axis=0, keepdims=True)


def _post(out_ap, den_ap, out_sp, den_sp, fc_w, fc_b):
    return pl.pallas_call(
        _post_body,
        grid=(GRID_TC,),
        in_specs=[
            pl.BlockSpec((2, ROWS_TC, D), lambda i: (0, i, 0)),
            pl.BlockSpec((2, ROWS_TC, DW), lambda i: (0, i, 0)),
            pl.BlockSpec((2, ROWS_TC, D), lambda i: (0, i, 0)),
            pl.BlockSpec((2, ROWS_TC, DW), lambda i: (0, i, 0)),
            pl.BlockSpec((D, D), lambda i: (0, 0)),
            pl.BlockSpec((D,), lambda i: (0,)),
        ],
        out_specs=[
            pl.BlockSpec((ROWS_TC, D), lambda i: (i, 0)),
            pl.BlockSpec((ROWS_TC, D), lambda i: (i, 0)),
            pl.BlockSpec((2, D), lambda i: (0, 0)),
        ],
        out_shape=[
            jax.ShapeDtypeStruct((N, D), jnp.float32),
            jax.ShapeDtypeStruct((N, D), jnp.float32),
            jax.ShapeDtypeStruct((2, D), jnp.float32),
        ],
    )(out_ap, den_ap, out_sp, den_sp, fc_w, fc_b)


def _combine_body(h_ap_ref, h_sp_ref, tsum_ref, sem_ref, out_ref):
    tmean = tsum_ref[...] * (1.0 / N)
    a = sem_ref[...][0]
    w0 = jnp.sum(tmean[0] * a)
    w1 = jnp.sum(tmean[1] * a)
    m = jnp.maximum(w0, w1)
    b0 = jnp.exp(w0 - m)
    b1 = jnp.exp(w1 - m)
    s = b0 + b1
    out_ref[...] = (b0 * h_ap_ref[...] + b1 * h_sp_ref[...]) / s


def _combine(h_ap, h_sp, tsum, attn_sem):
    return pl.pallas_call(
        _combine_body,
        grid=(GRID_TC,),
        in_specs=[
            pl.BlockSpec((ROWS_TC, D), lambda i: (i, 0)),
            pl.BlockSpec((ROWS_TC, D), lambda i: (i, 0)),
            pl.BlockSpec((2, D), lambda i: (0, 0)),
            pl.BlockSpec((1, D), lambda i: (0, 0)),
        ],
        out_specs=pl.BlockSpec((ROWS_TC, D), lambda i: (i, 0)),
        out_shape=jax.ShapeDtypeStruct((N, D), jnp.float32),
    )(h_ap, h_sp, tsum, attn_sem)


def _pad_chunks(x):
    return jnp.pad(x.reshape(NCHUNK, C), ((0, PAD_CHUNKS - NCHUNK), (0, 0)))


def kernel(feat_author, feat_subject, feat_paper, edge_index_ap, edge_index_sp,
           attn_l_ap, attn_r_ap, attn_l_sp, attn_r_sp, fc_w, fc_b, attn_sem):
    scal = _scalar_table(feat_author, feat_subject, feat_paper,
                         attn_l_ap, attn_r_ap, attn_l_sp, attn_r_sp)
    src_ap = _pad_chunks(edge_index_ap[0])
    dst_ap = _pad_chunks(edge_index_ap[1])
    src_sp = _pad_chunks(edge_index_sp[0])
    dst_sp = _pad_chunks(edge_index_sp[1])
    out_ap, den_ap, out_sp, den_sp = _sc_gat(scal, src_ap, dst_ap, src_sp, dst_sp,
                                             feat_author, feat_subject)
    h_ap, h_sp, tsum = _post(out_ap, den_ap, out_sp, den_sp, fc_w, fc_b)
    return _combine(h_ap, h_sp, tsum, attn_sem)
